# baseline (speedup 1.0000x reference)
import functools

import jax
import jax.numpy as jnp
from jax import lax
from jax.experimental import pallas as pl
from jax.experimental.pallas import tpu as pltpu
from jax.experimental.pallas import tpu_sc as plsc

B = 16384
D = 32
WD = 2 * D
NROW2 = 500000
PK = 2
BP = B // PK
NC, NS = 2, 16
NW = NC * NS
BPW = B // NW
PPW = BPW // PK
L = 16
NCHK = BPW // L


@functools.cache
def _make_sc_gather():
    mesh = plsc.VectorSubcoreMesh(
        core_axis_name="c", subcore_axis_name="s", num_cores=NC, num_subcores=NS
    )

    @functools.partial(
        pl.kernel,
        out_type=[
            jax.ShapeDtypeStruct((BP, PK * WD), jnp.float32),
            jax.ShapeDtypeStruct((BP, PK * WD), jnp.float32),
        ],
        mesh=mesh,
        scratch_types=[
            pltpu.VMEM((BPW,), jnp.int32),
            pltpu.VMEM((BPW,), jnp.int32),
            pltpu.VMEM((PPW, PK * WD), jnp.float32),
            pltpu.VMEM((PPW, PK * WD), jnp.float32),
            pltpu.SemaphoreType.DMA,
            pltpu.SemaphoreType.DMA,
        ],
    )
    def sc_gather(uidx_hbm, iidx_hbm, ut_hbm, it_hbm, u_out, i_out,
                  uidxv, iidxv, urows, irows, usem, isem):
        wid = lax.axis_index("s") * NC + lax.axis_index("c")
        pltpu.sync_copy(uidx_hbm.at[wid], uidxv)
        pltpu.sync_copy(iidx_hbm.at[wid], iidxv)

        def issue_body(j, _):
            ru = uidxv[pl.ds(j * L, L)]
            ri = iidxv[pl.ds(j * L, L)]
            for t in range(L):
                k = j * L + t
                q = k // PK
                p = k % PK
                pltpu.async_copy(
                    ut_hbm.at[ru[t] >> 1], urows.at[q, pl.ds(p * WD, WD)],
                    usem)
                pltpu.async_copy(
                    it_hbm.at[ri[t] >> 1], irows.at[q, pl.ds(p * WD, WD)],
                    isem)
            return 0

        lax.fori_loop(0, NCHK, issue_body, 0)

        def drain_body(j, _):
            for t in range(L):
                pltpu.make_async_copy(
                    ut_hbm.at[0], urows.at[0, pl.ds(0, WD)], usem).wait()
                pltpu.make_async_copy(
                    it_hbm.at[0], irows.at[0, pl.ds(0, WD)], isem).wait()
            return 0

        lax.fori_loop(0, NCHK, drain_body, 0)

        pltpu.sync_copy(urows, u_out.at[pl.ds(wid * PPW, PPW)])
        pltpu.sync_copy(irows, i_out.at[pl.ds(wid * PPW, PPW)])

    return sc_gather


PBLK = 1024


def _mlp_body(u_ref, i_ref, um_ref, im_ref, w1u_ref, w1i_ref, b1_ref,
              w2_ref, b2_ref, w3_ref, b3_ref, o_ref):
    half = (lax.broadcasted_iota(jnp.int32, (1, PK * WD), 1) // D) % PK
    um = um_ref[...]
    im = im_ref[...]
    um128 = jnp.concatenate(
        [jnp.broadcast_to(um[:, p:p + 1], (PBLK, WD)) for p in range(PK)],
        axis=1)
    im128 = jnp.concatenate(
        [jnp.broadcast_to(im[:, p:p + 1], (PBLK, WD)) for p in range(PK)],
        axis=1)
    mu = (half == um128).astype(jnp.float32)
    mi = (half == im128).astype(jnp.float32)
    h = jnp.dot(u_ref[...] * mu, w1u_ref[...],
                preferred_element_type=jnp.float32)
    h = h + jnp.dot(i_ref[...] * mi, w1i_ref[...],
                    preferred_element_type=jnp.float32)
    h = jnp.maximum(h + b1_ref[...], 0.0)
    h = jnp.dot(h, w2_ref[...], preferred_element_type=jnp.float32) + b2_ref[...]
    h = jnp.maximum(h, 0.0)
    z = jnp.dot(h, w3_ref[...], preferred_element_type=jnp.float32) + b3_ref[...]
    o_ref[...] = jax.nn.sigmoid(z)


def _mlp(u_g, i_g, u_m, i_m, w1u, w1i, b1, w2, b2, w3, b3):
    grid = (BP // PBLK,)
    full = lambda m: (0, 0)
    row = lambda m: (m, 0)
    return pl.pallas_call(
        _mlp_body,
        grid=grid,
        in_specs=[
            pl.BlockSpec((PBLK, PK * WD), row),
            pl.BlockSpec((PBLK, PK * WD), row),
            pl.BlockSpec((PBLK, PK), row),
            pl.BlockSpec((PBLK, PK), row),
            pl.BlockSpec(w1u.shape, full),
            pl.BlockSpec(w1i.shape, full),
            pl.BlockSpec(b1.shape, full),
            pl.BlockSpec(w2.shape, full),
            pl.BlockSpec(b2.shape, full),
            pl.BlockSpec(w3.shape, full),
            pl.BlockSpec(b3.shape, full),
        ],
        out_specs=pl.BlockSpec((PBLK, PK), row),
        out_shape=jax.ShapeDtypeStruct((BP, PK), jnp.float32),
        compiler_params=pltpu.CompilerParams(
            dimension_semantics=("arbitrary",),
        ),
    )(u_g, i_g, u_m, i_m, w1u, w1i, b1, w2, b2, w3, b3)


def kernel(user, item, user_table, item_table, W1, b1, W2, b2, W3, b3):
    user = user.astype(jnp.int32)
    item = item.astype(jnp.int32)
    u_g, i_g = _make_sc_gather()(
        user.reshape(NW, BPW), item.reshape(NW, BPW),
        user_table.reshape(NROW2, WD), item_table.reshape(NROW2, WD))
    eye = jnp.eye(PK, dtype=jnp.float32)
    w1u = jnp.kron(eye, jnp.tile(W1[:D], (PK, 1)))
    w1i = jnp.kron(eye, jnp.tile(W1[D:], (PK, 1)))
    w2 = jnp.kron(eye, W2)
    w3 = jnp.kron(eye, W3)
    b1t = jnp.tile(b1, (PK,)).reshape(1, -1)
    b2t = jnp.tile(b2, (PK,)).reshape(1, -1)
    b3t = jnp.tile(b3, (PK,)).reshape(1, -1)
    out = _mlp(u_g, i_g,
               (user % PK).reshape(BP, PK), (item % PK).reshape(BP, PK),
               w1u, w1i, b1t, w2, b2t, w3, b3t)
    return out.reshape(B, 1)


# speedup vs baseline: 1.6945x; 1.6945x over previous
import functools

import jax
import jax.numpy as jnp
from jax import lax
from jax.experimental import pallas as pl
from jax.experimental.pallas import tpu as pltpu
from jax.experimental.pallas import tpu_sc as plsc

B = 16384
D = 32
PK = 4
BP = B // PK
NC, NS = 2, 16
NW = NC * NS
BPW = B // NW
PPW = BPW // PK
L = 16
NCHK = BPW // L


@functools.cache
def _make_sc_gather():
    mesh = plsc.VectorSubcoreMesh(
        core_axis_name="c", subcore_axis_name="s", num_cores=NC, num_subcores=NS
    )

    @functools.partial(
        pl.kernel,
        out_type=[
            jax.ShapeDtypeStruct((BP, PK * D), jnp.float32),
            jax.ShapeDtypeStruct((BP, PK * D), jnp.float32),
        ],
        mesh=mesh,
        scratch_types=[
            pltpu.VMEM((BPW,), jnp.int32),
            pltpu.VMEM((BPW,), jnp.int32),
            pltpu.VMEM((PPW, PK * D), jnp.float32),
            pltpu.VMEM((PPW, PK * D), jnp.float32),
            pltpu.SemaphoreType.DMA,
            pltpu.SemaphoreType.DMA,
        ],
    )
    def sc_gather(uidx_hbm, iidx_hbm, ut_hbm, it_hbm, u_out, i_out,
                  uidxv, iidxv, urows, irows, usem, isem):
        wid = lax.axis_index("s") * NC + lax.axis_index("c")
        pltpu.sync_copy(uidx_hbm.at[wid], uidxv)
        pltpu.sync_copy(iidx_hbm.at[wid], iidxv)

        def issue_body(j, _):
            ru = uidxv[pl.ds(j * L, L)]
            ri = iidxv[pl.ds(j * L, L)]
            for t in range(L):
                k = j * L + t
                q = k // PK
                p = k % PK
                pltpu.async_copy(
                    ut_hbm.at[ru[t]], urows.at[q, pl.ds(p * D, D)], usem)
                pltpu.async_copy(
                    it_hbm.at[ri[t]], irows.at[q, pl.ds(p * D, D)], isem)
            return 0

        lax.fori_loop(0, NCHK, issue_body, 0)

        def drain_body(j, _):
            for t in range(L):
                pltpu.make_async_copy(
                    ut_hbm.at[0], urows.at[0, pl.ds(0, D)], usem).wait()
                pltpu.make_async_copy(
                    it_hbm.at[0], irows.at[0, pl.ds(0, D)], isem).wait()
            return 0

        lax.fori_loop(0, NCHK, drain_body, 0)

        pltpu.sync_copy(urows, u_out.at[pl.ds(wid * PPW, PPW)])
        pltpu.sync_copy(irows, i_out.at[pl.ds(wid * PPW, PPW)])

    return sc_gather


PBLK = 512


def _mlp_body(u_ref, i_ref, w1u_ref, w1i_ref, b1_ref,
              w2_ref, b2_ref, w3_ref, b3_ref, o_ref):
    h = jnp.dot(u_ref[...], w1u_ref[...], preferred_element_type=jnp.float32)
    h = h + jnp.dot(i_ref[...], w1i_ref[...], preferred_element_type=jnp.float32)
    h = jnp.maximum(h + b1_ref[...], 0.0)
    h = jnp.dot(h, w2_ref[...], preferred_element_type=jnp.float32) + b2_ref[...]
    h = jnp.maximum(h, 0.0)
    z = jnp.dot(h, w3_ref[...], preferred_element_type=jnp.float32) + b3_ref[...]
    o_ref[...] = jax.nn.sigmoid(z)


def _mlp(u_g, i_g, w1u, w1i, b1, w2, b2, w3, b3):
    grid = (BP // PBLK,)
    full = lambda m: (0, 0)
    row = lambda m: (m, 0)
    return pl.pallas_call(
        _mlp_body,
        grid=grid,
        in_specs=[
            pl.BlockSpec((PBLK, PK * D), row),
            pl.BlockSpec((PBLK, PK * D), row),
            pl.BlockSpec(w1u.shape, full),
            pl.BlockSpec(w1i.shape, full),
            pl.BlockSpec(b1.shape, full),
            pl.BlockSpec(w2.shape, full),
            pl.BlockSpec(b2.shape, full),
            pl.BlockSpec(w3.shape, full),
            pl.BlockSpec(b3.shape, full),
        ],
        out_specs=pl.BlockSpec((PBLK, PK), row),
        out_shape=jax.ShapeDtypeStruct((BP, PK), jnp.float32),
        compiler_params=pltpu.CompilerParams(
            dimension_semantics=("arbitrary",),
        ),
    )(u_g, i_g, w1u, w1i, b1, w2, b2, w3, b3)


def kernel(user, item, user_table, item_table, W1, b1, W2, b2, W3, b3):
    user = user.astype(jnp.int32)
    item = item.astype(jnp.int32)
    u_g, i_g = _make_sc_gather()(
        user.reshape(NW, BPW), item.reshape(NW, BPW), user_table, item_table)
    eye = jnp.eye(PK, dtype=jnp.float32)
    w1u = jnp.kron(eye, W1[:D])
    w1i = jnp.kron(eye, W1[D:])
    w2 = jnp.kron(eye, W2)
    w3 = jnp.kron(eye, W3)
    b1t = jnp.tile(b1, (PK,)).reshape(1, -1)
    b2t = jnp.tile(b2, (PK,)).reshape(1, -1)
    b3t = jnp.tile(b3, (PK,)).reshape(1, -1)
    out = _mlp(u_g, i_g, w1u, w1i, b1t, w2, b2t, w3, b3t)
    return out.reshape(B, 1)
